# baseline (speedup 1.0000x reference)
import functools

import jax
import jax.numpy as jnp
from jax import lax
from jax.experimental import pallas as pl
from jax.experimental.pallas import tpu as pltpu
from jax.experimental.pallas import tpu_sc as plsc

R = 16384
D = 1024
BR = 1024
NSTEP = R // BR
K = R // 2

NT = 16
VPT = R // NT
NCH = VPT // 16
ITERS = 32


def _rowsum_body(pred_ref, target_ref, out_ref):
    i = pl.program_id(0)
    a = pred_ref[...]
    b = target_ref[...]
    row = jnp.sum(jnp.abs(a - b), axis=1)
    out_ref[pl.ds(i, 1), :] = row.reshape(1, BR)


def _rowsums(pred, target):
    return pl.pallas_call(
        _rowsum_body,
        grid=(NSTEP,),
        in_specs=[
            pl.BlockSpec((BR, D), lambda i: (i, 0)),
            pl.BlockSpec((BR, D), lambda i: (i, 0)),
        ],
        out_specs=pl.BlockSpec((NSTEP, BR), lambda i: (0, 0)),
        out_shape=jax.ShapeDtypeStruct((NSTEP, BR), jnp.float32),
    )(pred, target)


def _sc_body(rows_hbm, out_hbm, vals, xiv, stage, stagef, allc, allc2,
             sh_cnt):
    cid = lax.axis_index("c")
    sid = lax.axis_index("s")

    ones_i = jnp.ones((16,), jnp.int32)
    zeros_i = jnp.zeros((16,), jnp.int32)
    zeros_f = jnp.zeros((16,), jnp.float32)

    @pl.when(cid == 0)
    def _work():
        pltpu.sync_copy(rows_hbm.at[pl.ds(sid * VPT, VPT)], vals)
        for j in range(NCH):
            xiv[pl.ds(j * 16, 16)] = plsc.bitcast(
                vals[pl.ds(j * 16, 16)], jnp.int32)

        def bis(i, carry):
            lo, hi = carry
            mid = lo + ((hi - lo + jnp.int32(1)) >> 1)
            midv = jnp.broadcast_to(mid, (16,))
            cnt = zeros_i
            for j in range(NCH):
                cnt = cnt + jnp.where(
                    xiv[pl.ds(j * 16, 16)] >= midv, ones_i, zeros_i)
            stage[...] = cnt
            pltpu.sync_copy(stage, sh_cnt.at[i, sid])
            plsc.subcore_barrier()
            pltpu.sync_copy(sh_cnt.at[i], allc)
            tot = zeros_i
            for r in range(NT):
                tot = tot + allc[r]
            tot_s = jnp.sum(tot)
            take = tot_s >= K
            return (jnp.where(take, mid, lo),
                    jnp.where(take, hi, mid - jnp.int32(1)))

        lo, _ = lax.fori_loop(0, ITERS, bis,
                              (jnp.int32(0), jnp.int32(0x7F800000)))

        vf = plsc.bitcast(jnp.broadcast_to(lo, (16,)), jnp.float32)
        cg = zeros_i
        sg = zeros_f
        for j in range(NCH):
            xj = vals[pl.ds(j * 16, 16)]
            m = xj > vf
            cg = cg + jnp.where(m, ones_i, zeros_i)
            sg = sg + jnp.where(m, xj, zeros_f)
        stage[...] = cg
        pltpu.sync_copy(stage, sh_cnt.at[ITERS, sid])
        stage[...] = plsc.bitcast(sg, jnp.int32)
        pltpu.sync_copy(stage, sh_cnt.at[ITERS + 1, sid])
        plsc.subcore_barrier()

        pltpu.sync_copy(sh_cnt.at[pl.ds(ITERS, 2)], allc2)
        cnti = zeros_i
        totf = zeros_f
        for r in range(NT):
            cnti = cnti + allc2[0, r]
            totf = totf + plsc.bitcast(allc2[1, r], jnp.float32)
        kv = jnp.full((16,), float(K), jnp.float32)
        cnt_v = jnp.broadcast_to(jnp.sum(cnti).astype(jnp.float32), (16,))
        sum_v = jnp.broadcast_to(jnp.sum(totf), (16,))
        resv = (sum_v + (kv - cnt_v) * vf) / kv

        @pl.when(sid == 0)
        def _fin():
            stagef[...] = resv
            pltpu.sync_copy(stagef, out_hbm)


def _topk_mean_sc(rows_flat):
    mesh = plsc.VectorSubcoreMesh(
        core_axis_name="c", subcore_axis_name="s",
        num_cores=2, num_subcores=16)
    fn = pl.kernel(
        _sc_body,
        out_type=jax.ShapeDtypeStruct((16,), jnp.float32),
        mesh=mesh,
        compiler_params=pltpu.CompilerParams(needs_layout_passes=False),
        scratch_types=[
            pltpu.VMEM((VPT,), jnp.float32),
            pltpu.VMEM((VPT,), jnp.int32),
            pltpu.VMEM((16,), jnp.int32),
            pltpu.VMEM((16,), jnp.float32),
            pltpu.VMEM((NT, 16), jnp.int32),
            pltpu.VMEM((2, NT, 16), jnp.int32),
            pltpu.VMEM_SHARED((ITERS + 2, NT, 16), jnp.int32),
        ],
    )
    return fn(rows_flat)


def kernel(pred, target):
    p = pred.reshape(R, D)
    t = target.reshape(R, D)
    rows = _rowsums(p, t)
    out = _topk_mean_sc(rows.reshape(R))
    return out[0]


# speedup vs baseline: 1.6189x; 1.6189x over previous
import jax
import jax.numpy as jnp
from jax.experimental import pallas as pl
from jax.experimental.pallas import tpu as pltpu

R = 16384
D = 1024
BR = 1024
NSTEP = R // BR
K = R // 2


def _body(pred_ref, target_ref, out_ref, acc_ref):
    i = pl.program_id(0)
    a = pred_ref[...]
    b = target_ref[...]
    row = jnp.sum(jnp.abs(a - b), axis=1)
    acc_ref[pl.ds(i, 1), :] = row.reshape(1, BR)

    @pl.when(i == NSTEP - 1)
    def _finalize():
        x = acc_ref[...]
        xi = jax.lax.bitcast_convert_type(x, jnp.int32)

        def bisect(carry):
            lo, hi = carry
            mid = lo + ((hi - lo + 1) >> 1)
            cnt = jnp.sum((xi >= mid).astype(jnp.int32))
            take = cnt >= K
            return (jnp.where(take, mid, lo), jnp.where(take, hi, mid - 1))

        lo0 = jax.lax.bitcast_convert_type(jnp.min(x), jnp.int32)
        hi0 = jax.lax.bitcast_convert_type(jnp.max(x), jnp.int32)
        lo, _ = jax.lax.while_loop(
            lambda c: c[0] < c[1], bisect, (lo0, hi0))
        v = jax.lax.bitcast_convert_type(lo, jnp.float32)

        gt = x > v
        cnt_gt = jnp.sum(gt.astype(jnp.int32))
        sum_gt = jnp.sum(jnp.where(gt, x, 0.0))
        res = (sum_gt + (K - cnt_gt).astype(jnp.float32) * v) / K
        out_ref[...] = jnp.broadcast_to(res, (1, 1))


def kernel(pred, target):
    p = pred.reshape(R, D)
    t = target.reshape(R, D)
    out = pl.pallas_call(
        _body,
        grid=(NSTEP,),
        in_specs=[
            pl.BlockSpec((BR, D), lambda i: (i, 0)),
            pl.BlockSpec((BR, D), lambda i: (i, 0)),
        ],
        out_specs=pl.BlockSpec((1, 1), lambda i: (0, 0)),
        out_shape=jax.ShapeDtypeStruct((1, 1), jnp.float32),
        scratch_shapes=[pltpu.VMEM((NSTEP, BR), jnp.float32)],
    )(p, t)
    return out[0, 0]
